# baseline (speedup 1.0000x reference)
import functools
import jax
import jax.numpy as jnp
from jax import lax
from jax.experimental import pallas as pl
from jax.experimental.pallas import tpu as pltpu

N_NODES = 10000
E_EDGES = 320000
DN = 128
DE = 16
DA = 128
HEADS = 4

_INTERPRET = False


def _ln(x, g, b, eps=1e-5):
    m = jnp.mean(x, axis=-1, keepdims=True)
    v = jnp.mean((x - m) ** 2, axis=-1, keepdims=True)
    return (x - m) / jnp.sqrt(v + eps) * g + b


def _node_pre_body(x_ref, m_ref, b_ref, o_ref):
    o_ref[...] = (
        jnp.dot(x_ref[...], m_ref[...], preferred_element_type=jnp.float32)
        + b_ref[...]
    )


def _node_precompute(x, M, b):
    n = x.shape[0]
    BLK = 512
    grid = (pl.cdiv(n, BLK),)
    return pl.pallas_call(
        _node_pre_body,
        grid=grid,
        in_specs=[
            pl.BlockSpec((BLK, DN), lambda i: (i, 0)),
            pl.BlockSpec((DN, 576), lambda i: (0, 0)),
            pl.BlockSpec((1, 576), lambda i: (0, 0)),
        ],
        out_specs=pl.BlockSpec((BLK, 576), lambda i: (i, 0)),
        out_shape=jax.ShapeDtypeStruct((n, 576), jnp.float32),
        interpret=_INTERPRET,
    )(x, M, b)


def _edge_body(gr_ref, gc_ref, ef_ref, rev_ref,
               w1bt_ref, w1ct_ref, bne1_ref, w2t_ref, bne2_ref,
               geln_ref, beln_ref,
               wg1t_ref, bg1_ref, bng_ref, bnb_ref, wg2t_ref, bg2_ref,
               wet_ref, be_ref, m1t_ref, ba1_ref, m2t_ref, ba2_ref, s_ref,
               efo_ref, xxe_ref):
    ef = ef_ref[...]
    h = jax.nn.relu(jnp.dot(ef, wg1t_ref[...], preferred_element_type=jnp.float32)
                    + bg1_ref[...])
    h = h * (bng_ref[...] / jnp.sqrt(1.0 + 1e-5)) + bnb_ref[...]
    gates = jax.nn.sigmoid(jnp.dot(h, wg2t_ref[...], preferred_element_type=jnp.float32)
                           + bg2_ref[...])
    rev = gates * rev_ref[...]
    g1 = gr_ref[:, :160]
    gq = gr_ref[:, 160:288]
    g2 = gc_ref[:, :160]
    gv = gc_ref[:, 160:288]
    mid = g1 + g2 + bne1_ref[...]
    mid = mid + jnp.dot(ef, w1bt_ref[...], preferred_element_type=jnp.float32)
    mid = mid + jnp.dot(rev, w1ct_ref[...], preferred_element_type=jnp.float32)
    mid = jax.nn.relu(mid)
    efo = jnp.dot(mid, w2t_ref[...], preferred_element_type=jnp.float32) + bne2_ref[...]
    efo_ref[...] = _ln(efo, geln_ref[...], beln_ref[...])
    eh = jnp.dot(ef, wet_ref[...], preferred_element_type=jnp.float32) + be_ref[...]
    qe = jnp.concatenate([gq, eh], axis=1)
    a = jax.nn.relu(jnp.dot(qe, m1t_ref[...], preferred_element_type=jnp.float32)
                    + ba1_ref[...])
    p = jnp.dot(a, m2t_ref[...], preferred_element_type=jnp.float32) + ba2_ref[...]
    ex = jnp.exp(p)
    den = jnp.dot(ex, s_ref[...], preferred_element_type=jnp.float32)
    inv = jnp.dot(1.0 / den, s_ref[...].T, preferred_element_type=jnp.float32)
    xxe_ref[...] = ex * inv * gv


def _edge_compute(gr, gc, ef, rev, wdict):
    e = ef.shape[0]
    BLK = 512
    grid = (pl.cdiv(e, BLK),)
    full = lambda shp: pl.BlockSpec(shp, lambda i: tuple(0 for _ in shp))
    in_specs = [
        pl.BlockSpec((BLK, 288), lambda i: (i, 0)),
        pl.BlockSpec((BLK, 288), lambda i: (i, 0)),
        pl.BlockSpec((BLK, DE), lambda i: (i, 0)),
        pl.BlockSpec((BLK, DE), lambda i: (i, 0)),
        full((DE, 160)), full((DE, 160)), full((1, 160)),
        full((160, DE)), full((1, DE)),
        full((1, DE)), full((1, DE)),
        full((DE, 8)), full((1, 8)), full((1, 8)), full((1, 8)),
        full((8, 1)), full((1, 1)),
        full((DE, DE)), full((1, DE)),
        full((144, 144)), full((1, 144)),
        full((144, 128)), full((1, 128)),
        full((128, 4)),
    ]
    return pl.pallas_call(
        _edge_body,
        grid=grid,
        in_specs=in_specs,
        out_specs=[
            pl.BlockSpec((BLK, DE), lambda i: (i, 0)),
            pl.BlockSpec((BLK, DA), lambda i: (i, 0)),
        ],
        out_shape=[
            jax.ShapeDtypeStruct((e, DE), jnp.float32),
            jax.ShapeDtypeStruct((e, DA), jnp.float32),
        ],
        interpret=_INTERPRET,
    )(gr, gc, ef, rev, *wdict)


def _node_out_body(x_ref, agg_ref, ss_ref, os_ref, ds_ref, do_ref,
                   wtt_ref, bt_ref, wp1t_ref, bp1_ref, wp2t_ref, bp2_ref,
                   gln_ref, bln_ref, o_ref):
    ds = ds_ref[...]
    do = do_ref[...]
    subj = jnp.where(ds > 0, ss_ref[...] / jnp.maximum(ds, 1.0), 0.0)
    obj = jnp.where(do > 0, os_ref[...] / jnp.maximum(do, 1.0), 0.0)
    twin = (jnp.dot(jnp.concatenate([subj, obj], axis=1), wtt_ref[...],
                    preferred_element_type=jnp.float32) + bt_ref[...])
    agg = jnp.where(ds > 0, agg_ref[...], 0.0)
    xx = jax.nn.relu(agg) * jax.nn.sigmoid(twin)
    cat = jnp.concatenate([x_ref[...], xx], axis=1)
    hh = jax.nn.relu(jnp.dot(cat, wp1t_ref[...], preferred_element_type=jnp.float32)
                     + bp1_ref[...])
    out = jnp.dot(hh, wp2t_ref[...], preferred_element_type=jnp.float32) + bp2_ref[...]
    o_ref[...] = _ln(out, gln_ref[...], bln_ref[...])


def _node_out(x, agg, ss, os_, ds, do, wlist):
    n = x.shape[0]
    BLK = 512
    grid = (pl.cdiv(n, BLK),)
    full = lambda shp: pl.BlockSpec(shp, lambda i: tuple(0 for _ in shp))
    return pl.pallas_call(
        _node_out_body,
        grid=grid,
        in_specs=[
            pl.BlockSpec((BLK, DN), lambda i: (i, 0)),
            pl.BlockSpec((BLK, DA), lambda i: (i, 0)),
            pl.BlockSpec((BLK, DE), lambda i: (i, 0)),
            pl.BlockSpec((BLK, DE), lambda i: (i, 0)),
            pl.BlockSpec((BLK, 1), lambda i: (i, 0)),
            pl.BlockSpec((BLK, 1), lambda i: (i, 0)),
            full((32, 128)), full((1, 128)),
            full((256, 256)), full((1, 256)),
            full((256, 128)), full((1, 128)),
            full((1, 128)), full((1, 128)),
        ],
        out_specs=pl.BlockSpec((BLK, DN), lambda i: (i, 0)),
        out_shape=jax.ShapeDtypeStruct((n, DN), jnp.float32),
        interpret=_INTERPRET,
    )(x, agg, ss, os_, ds, do, *wlist)


def kernel(x, edge_feature, edge_index, W_ne1, b_ne1, W_ne2, b_ne2, g_eln,
           b_eln, Wg1, bg1, bn_g, bn_b, Wg2, bg2, Wv, bv, Wq, bq, We, be,
           Wa1, ba1, Wa2, ba2, Wt, bt, Wp1, bp1, Wp2, bp2, g_ln, b_ln):
    N = x.shape[0]
    E = edge_index.shape[1]
    row = edge_index[0].astype(jnp.int32)
    col = edge_index[1].astype(jnp.int32)

    keys = row.astype(jnp.int64) * N + col.astype(jnp.int64)
    rkeys = col.astype(jnp.int64) * N + row.astype(jnp.int64)
    order = jnp.argsort(keys)
    sk = keys[order]
    pos = jnp.clip(jnp.searchsorted(sk, rkeys), 0, E - 1)
    found = sk[pos] == rkeys
    rev_idx = order[pos]
    rev_raw = jnp.where(found[:, None], edge_feature[rev_idx], 0.0)

    M = jnp.concatenate([
        W_ne1[:, :DN].T, Wq.T, W_ne1[:, 160:288].T, Wv.T], axis=1)
    bnode = jnp.concatenate([
        jnp.zeros((160,), jnp.float32), bq,
        jnp.zeros((160,), jnp.float32), bv])[None, :]
    nodes = _node_precompute(x, M, bnode)
    gr = nodes[:, :288][row]
    gc = nodes[:, 288:][col]

    eye4 = jnp.eye(4, dtype=jnp.float32)
    wlist = (
        W_ne1[:, 128:144].T, W_ne1[:, 144:160].T, b_ne1[None, :],
        W_ne2.T, b_ne2[None, :], g_eln[None, :], b_eln[None, :],
        Wg1.T, bg1[None, :], bn_g[None, :], bn_b[None, :],
        Wg2.T, bg2[None, :],
        We.T, be[None, :],
        jnp.kron(Wa1, eye4).T, jnp.kron(ba1, jnp.ones(4, jnp.float32))[None, :],
        jnp.kron(Wa2, eye4).T, jnp.kron(ba2, jnp.ones(4, jnp.float32))[None, :],
        jnp.tile(eye4, (32, 1)),
    )
    ef_out, xx_e = _edge_compute(gr, gc, edge_feature, rev_raw, wlist)

    agg = jax.ops.segment_max(xx_e, row, num_segments=N)
    ones_e = jnp.ones((E,), jnp.float32)
    deg_s = jax.ops.segment_sum(ones_e, row, num_segments=N)
    deg_o = jax.ops.segment_sum(ones_e, col, num_segments=N)
    subj_sum = jax.ops.segment_sum(ef_out, row, num_segments=N)
    obj_sum = jax.ops.segment_sum(ef_out, col, num_segments=N)
    agg = jnp.where(deg_s[:, None] > 0, agg, 0.0)

    wlist3 = (
        Wt.T, bt[None, :], Wp1.T, bp1[None, :], Wp2.T, bp2[None, :],
        g_ln[None, :], b_ln[None, :],
    )
    out = _node_out(x, agg, subj_sum, obj_sum,
                    deg_s[:, None], deg_o[:, None], wlist3)
    return (out, ef_out)


# speedup vs baseline: 1.1766x; 1.1766x over previous
import functools
import jax
import jax.numpy as jnp
from jax import lax
from jax.experimental import pallas as pl
from jax.experimental.pallas import tpu as pltpu
from jax.experimental.pallas import tpu_sc as plsc

N_NODES = 10000
E_EDGES = 320000
DN = 128
DE = 16
DA = 128
HEADS = 4

_INTERPRET = False


def _ln(x, g, b, eps=1e-5):
    m = jnp.mean(x, axis=-1, keepdims=True)
    v = jnp.mean((x - m) ** 2, axis=-1, keepdims=True)
    return (x - m) / jnp.sqrt(v + eps) * g + b


def _node_pre_body(x_ref, m_ref, b_ref, or_ref, oc_ref):
    res = (
        jnp.dot(x_ref[...], m_ref[...], preferred_element_type=jnp.float32)
        + b_ref[...]
    )
    or_ref[...] = res[:, :384]
    oc_ref[...] = res[:, 384:]


def _node_precompute(x, M, b):
    n = x.shape[0]
    BLK = 512
    grid = (pl.cdiv(n, BLK),)
    return pl.pallas_call(
        _node_pre_body,
        grid=grid,
        in_specs=[
            pl.BlockSpec((BLK, DN), lambda i: (i, 0)),
            pl.BlockSpec((DN, 768), lambda i: (0, 0)),
            pl.BlockSpec((1, 768), lambda i: (0, 0)),
        ],
        out_specs=[
            pl.BlockSpec((BLK, 384), lambda i: (i, 0)),
            pl.BlockSpec((BLK, 384), lambda i: (i, 0)),
        ],
        out_shape=[
            jax.ShapeDtypeStruct((n, 384), jnp.float32),
            jax.ShapeDtypeStruct((n, 384), jnp.float32),
        ],
        interpret=_INTERPRET,
    )(x, M, b)


def _sc_gather2(nodes_r, nodes_c, row, col):
    E = row.shape[0]
    K = 128
    CH = E // K
    NW = 32
    ITERS = pl.cdiv(CH, NW)
    mesh = plsc.VectorSubcoreMesh(core_axis_name="c", subcore_axis_name="s")

    @functools.partial(
        pl.kernel, mesh=mesh,
        out_type=[
            jax.ShapeDtypeStruct((E, 384), jnp.float32),
            jax.ShapeDtypeStruct((E, 384), jnp.float32),
        ],
        scratch_types=[
            pltpu.VMEM((K,), jnp.int32), pltpu.VMEM((K,), jnp.int32),
            pltpu.VMEM((K, 384), jnp.float32),
            pltpu.VMEM((K, 384), jnp.float32),
            pltpu.SemaphoreType.DMA, pltpu.SemaphoreType.DMA,
        ],
    )
    def k(nr_hbm, nc_hbm, row_hbm, col_hbm, gr_hbm, gc_hbm,
          idxr, idxc, bufr, bufc, sem1, sem2):
        wid = lax.axis_index("s") * 2 + lax.axis_index("c")

        def body(i, carry):
            c = wid + i * NW

            @pl.when(c < CH)
            def _():
                base = c * K
                pltpu.sync_copy(row_hbm.at[pl.ds(base, K)], idxr)
                pltpu.sync_copy(col_hbm.at[pl.ds(base, K)], idxc)
                cp1 = pltpu.async_copy(nr_hbm.at[idxr], bufr, sem1)
                cp2 = pltpu.async_copy(nc_hbm.at[idxc], bufc, sem2)
                cp1.wait()
                cp2.wait()
                pltpu.sync_copy(bufr, gr_hbm.at[pl.ds(base, K)])
                pltpu.sync_copy(bufc, gc_hbm.at[pl.ds(base, K)])
            return carry

        lax.fori_loop(0, ITERS, body, 0)

    return k(nodes_r, nodes_c, row, col)


def _edge_body(gr_ref, gc_ref, ef_ref, rev_ref,
               w1bt_ref, w1ct_ref, bne1_ref, w2t_ref, bne2_ref,
               geln_ref, beln_ref,
               wg1t_ref, bg1_ref, bng_ref, bnb_ref, wg2t_ref, bg2_ref,
               wet_ref, be_ref, m1t_ref, ba1_ref, m2t_ref, ba2_ref, s_ref,
               efo_ref, xxe_ref):
    ef = ef_ref[...]
    h = jax.nn.relu(jnp.dot(ef, wg1t_ref[...], preferred_element_type=jnp.float32)
                    + bg1_ref[...])
    h = h * (bng_ref[...] / jnp.sqrt(1.0 + 1e-5)) + bnb_ref[...]
    gates = jax.nn.sigmoid(jnp.dot(h, wg2t_ref[...], preferred_element_type=jnp.float32)
                           + bg2_ref[...])
    rev = gates * rev_ref[...]
    g1 = gr_ref[:, :160]
    gq = gr_ref[:, 160:288]
    g2 = gc_ref[:, :160]
    gv = gc_ref[:, 160:288]
    mid = g1 + g2 + bne1_ref[...]
    mid = mid + jnp.dot(ef, w1bt_ref[...], preferred_element_type=jnp.float32)
    mid = mid + jnp.dot(rev, w1ct_ref[...], preferred_element_type=jnp.float32)
    mid = jax.nn.relu(mid)
    efo = jnp.dot(mid, w2t_ref[...], preferred_element_type=jnp.float32) + bne2_ref[...]
    efo_ref[...] = _ln(efo, geln_ref[...], beln_ref[...])
    eh = jnp.dot(ef, wet_ref[...], preferred_element_type=jnp.float32) + be_ref[...]
    qe = jnp.concatenate([gq, eh], axis=1)
    a = jax.nn.relu(jnp.dot(qe, m1t_ref[...], preferred_element_type=jnp.float32)
                    + ba1_ref[...])
    p = jnp.dot(a, m2t_ref[...], preferred_element_type=jnp.float32) + ba2_ref[...]
    ex = jnp.exp(p)
    den = jnp.dot(ex, s_ref[...], preferred_element_type=jnp.float32)
    inv = jnp.dot(1.0 / den, s_ref[...].T, preferred_element_type=jnp.float32)
    xxe_ref[...] = ex * inv * gv


def _edge_compute(gr, gc, ef, rev, wdict):
    e = ef.shape[0]
    BLK = 512
    grid = (pl.cdiv(e, BLK),)
    full = lambda shp: pl.BlockSpec(shp, lambda i: tuple(0 for _ in shp))
    in_specs = [
        pl.BlockSpec((BLK, 384), lambda i: (i, 0)),
        pl.BlockSpec((BLK, 384), lambda i: (i, 0)),
        pl.BlockSpec((BLK, DE), lambda i: (i, 0)),
        pl.BlockSpec((BLK, DE), lambda i: (i, 0)),
        full((DE, 160)), full((DE, 160)), full((1, 160)),
        full((160, DE)), full((1, DE)),
        full((1, DE)), full((1, DE)),
        full((DE, 8)), full((1, 8)), full((1, 8)), full((1, 8)),
        full((8, 1)), full((1, 1)),
        full((DE, DE)), full((1, DE)),
        full((144, 144)), full((1, 144)),
        full((144, 128)), full((1, 128)),
        full((128, 4)),
    ]
    return pl.pallas_call(
        _edge_body,
        grid=grid,
        in_specs=in_specs,
        out_specs=[
            pl.BlockSpec((BLK, DE), lambda i: (i, 0)),
            pl.BlockSpec((BLK, DA), lambda i: (i, 0)),
        ],
        out_shape=[
            jax.ShapeDtypeStruct((e, DE), jnp.float32),
            jax.ShapeDtypeStruct((e, DA), jnp.float32),
        ],
        interpret=_INTERPRET,
    )(gr, gc, ef, rev, *wdict)


def _node_out_body(x_ref, agg_ref, ss_ref, os_ref, ds_ref, do_ref,
                   wtt_ref, bt_ref, wp1t_ref, bp1_ref, wp2t_ref, bp2_ref,
                   gln_ref, bln_ref, o_ref):
    ds = ds_ref[...]
    do = do_ref[...]
    subj = jnp.where(ds > 0, ss_ref[...] / jnp.maximum(ds, 1.0), 0.0)
    obj = jnp.where(do > 0, os_ref[...] / jnp.maximum(do, 1.0), 0.0)
    twin = (jnp.dot(jnp.concatenate([subj, obj], axis=1), wtt_ref[...],
                    preferred_element_type=jnp.float32) + bt_ref[...])
    agg = jnp.where(ds > 0, agg_ref[...], 0.0)
    xx = jax.nn.relu(agg) * jax.nn.sigmoid(twin)
    cat = jnp.concatenate([x_ref[...], xx], axis=1)
    hh = jax.nn.relu(jnp.dot(cat, wp1t_ref[...], preferred_element_type=jnp.float32)
                     + bp1_ref[...])
    out = jnp.dot(hh, wp2t_ref[...], preferred_element_type=jnp.float32) + bp2_ref[...]
    o_ref[...] = _ln(out, gln_ref[...], bln_ref[...])


def _node_out(x, agg, ss, os_, ds, do, wlist):
    n = x.shape[0]
    BLK = 512
    grid = (pl.cdiv(n, BLK),)
    full = lambda shp: pl.BlockSpec(shp, lambda i: tuple(0 for _ in shp))
    return pl.pallas_call(
        _node_out_body,
        grid=grid,
        in_specs=[
            pl.BlockSpec((BLK, DN), lambda i: (i, 0)),
            pl.BlockSpec((BLK, DA), lambda i: (i, 0)),
            pl.BlockSpec((BLK, DE), lambda i: (i, 0)),
            pl.BlockSpec((BLK, DE), lambda i: (i, 0)),
            pl.BlockSpec((BLK, 1), lambda i: (i, 0)),
            pl.BlockSpec((BLK, 1), lambda i: (i, 0)),
            full((32, 128)), full((1, 128)),
            full((256, 256)), full((1, 256)),
            full((256, 128)), full((1, 128)),
            full((1, 128)), full((1, 128)),
        ],
        out_specs=pl.BlockSpec((BLK, DN), lambda i: (i, 0)),
        out_shape=jax.ShapeDtypeStruct((n, DN), jnp.float32),
        interpret=_INTERPRET,
    )(x, agg, ss, os_, ds, do, *wlist)


def kernel(x, edge_feature, edge_index, W_ne1, b_ne1, W_ne2, b_ne2, g_eln,
           b_eln, Wg1, bg1, bn_g, bn_b, Wg2, bg2, Wv, bv, Wq, bq, We, be,
           Wa1, ba1, Wa2, ba2, Wt, bt, Wp1, bp1, Wp2, bp2, g_ln, b_ln):
    N = x.shape[0]
    E = edge_index.shape[1]
    row = edge_index[0].astype(jnp.int32)
    col = edge_index[1].astype(jnp.int32)

    keys = row.astype(jnp.int64) * N + col.astype(jnp.int64)
    rkeys = col.astype(jnp.int64) * N + row.astype(jnp.int64)
    order = jnp.argsort(keys)
    sk = keys[order]
    pos = jnp.clip(jnp.searchsorted(sk, rkeys), 0, E - 1)
    found = sk[pos] == rkeys
    rev_idx = order[pos]
    rev_raw = jnp.where(found[:, None], edge_feature[rev_idx], 0.0)

    z96 = jnp.zeros((DN, 96), jnp.float32)
    M = jnp.concatenate([
        W_ne1[:, :DN].T, Wq.T, z96, W_ne1[:, 160:288].T, Wv.T, z96], axis=1)
    bnode = jnp.concatenate([
        jnp.zeros((160,), jnp.float32), bq, jnp.zeros((96,), jnp.float32),
        jnp.zeros((160,), jnp.float32), bv,
        jnp.zeros((96,), jnp.float32)])[None, :]
    nodes_r, nodes_c = _node_precompute(x, M, bnode)
    gr, gc = _sc_gather2(nodes_r, nodes_c, row, col)

    eye4 = jnp.eye(4, dtype=jnp.float32)
    wlist = (
        W_ne1[:, 128:144].T, W_ne1[:, 144:160].T, b_ne1[None, :],
        W_ne2.T, b_ne2[None, :], g_eln[None, :], b_eln[None, :],
        Wg1.T, bg1[None, :], bn_g[None, :], bn_b[None, :],
        Wg2.T, bg2[None, :],
        We.T, be[None, :],
        jnp.kron(Wa1, eye4).T, jnp.kron(ba1, jnp.ones(4, jnp.float32))[None, :],
        jnp.kron(Wa2, eye4).T, jnp.kron(ba2, jnp.ones(4, jnp.float32))[None, :],
        jnp.tile(eye4, (32, 1)),
    )
    ef_out, xx_e = _edge_compute(gr, gc, edge_feature, rev_raw, wlist)

    agg = jax.ops.segment_max(xx_e, row, num_segments=N)
    ones_e = jnp.ones((E,), jnp.float32)
    deg_s = jax.ops.segment_sum(ones_e, row, num_segments=N)
    deg_o = jax.ops.segment_sum(ones_e, col, num_segments=N)
    subj_sum = jax.ops.segment_sum(ef_out, row, num_segments=N)
    obj_sum = jax.ops.segment_sum(ef_out, col, num_segments=N)
    agg = jnp.where(deg_s[:, None] > 0, agg, 0.0)

    wlist3 = (
        Wt.T, bt[None, :], Wp1.T, bp1[None, :], Wp2.T, bp2[None, :],
        g_ln[None, :], b_ln[None, :],
    )
    out = _node_out(x, agg, subj_sum, obj_sum,
                    deg_s[:, None], deg_o[:, None], wlist3)
    return (out, ef_out)


# speedup vs baseline: 1.3282x; 1.1289x over previous
import functools
import jax
import jax.numpy as jnp
from jax import lax
from jax.experimental import pallas as pl
from jax.experimental.pallas import tpu as pltpu
from jax.experimental.pallas import tpu_sc as plsc

N_NODES = 10000
E_EDGES = 320000
DN = 128
DE = 16
DA = 128
HEADS = 4

_INTERPRET = False


def _ln(x, g, b, eps=1e-5):
    m = jnp.mean(x, axis=-1, keepdims=True)
    v = jnp.mean((x - m) ** 2, axis=-1, keepdims=True)
    return (x - m) / jnp.sqrt(v + eps) * g + b


def _node_pre_body(x_ref, m_ref, b_ref, or_ref, oc_ref):
    res = (
        jnp.dot(x_ref[...], m_ref[...], preferred_element_type=jnp.float32)
        + b_ref[...]
    )
    or_ref[...] = res[:, :384]
    oc_ref[...] = res[:, 384:]


def _node_precompute(x, M, b):
    n = x.shape[0]
    BLK = 512
    grid = (pl.cdiv(n, BLK),)
    return pl.pallas_call(
        _node_pre_body,
        grid=grid,
        in_specs=[
            pl.BlockSpec((BLK, DN), lambda i: (i, 0)),
            pl.BlockSpec((DN, 768), lambda i: (0, 0)),
            pl.BlockSpec((1, 768), lambda i: (0, 0)),
        ],
        out_specs=[
            pl.BlockSpec((BLK, 384), lambda i: (i, 0)),
            pl.BlockSpec((BLK, 384), lambda i: (i, 0)),
        ],
        out_shape=[
            jax.ShapeDtypeStruct((n, 384), jnp.float32),
            jax.ShapeDtypeStruct((n, 384), jnp.float32),
        ],
        interpret=_INTERPRET,
    )(x, M, b)


def _sc_gather2(nodes_r, nodes_c, row, col):
    E = row.shape[0]
    K = 128
    CH = E // K
    NW = 32
    ITERS = pl.cdiv(CH, NW)
    mesh = plsc.VectorSubcoreMesh(core_axis_name="c", subcore_axis_name="s")

    @functools.partial(
        pl.kernel, mesh=mesh,
        out_type=[
            jax.ShapeDtypeStruct((E, 384), jnp.float32),
            jax.ShapeDtypeStruct((E, 384), jnp.float32),
        ],
        scratch_types=[
            pltpu.VMEM((K,), jnp.int32), pltpu.VMEM((K,), jnp.int32),
            pltpu.VMEM((K, 384), jnp.float32),
            pltpu.VMEM((K, 384), jnp.float32),
            pltpu.SemaphoreType.DMA, pltpu.SemaphoreType.DMA,
        ],
    )
    def k(nr_hbm, nc_hbm, row_hbm, col_hbm, gr_hbm, gc_hbm,
          idxr, idxc, bufr, bufc, sem1, sem2):
        wid = lax.axis_index("s") * 2 + lax.axis_index("c")

        def body(i, carry):
            c = wid + i * NW

            @pl.when(c < CH)
            def _():
                base = c * K
                pltpu.sync_copy(row_hbm.at[pl.ds(base, K)], idxr)
                pltpu.sync_copy(col_hbm.at[pl.ds(base, K)], idxc)
                cp1 = pltpu.async_copy(nr_hbm.at[idxr], bufr, sem1)
                cp2 = pltpu.async_copy(nc_hbm.at[idxc], bufc, sem2)
                cp1.wait()
                cp2.wait()
                pltpu.sync_copy(bufr, gr_hbm.at[pl.ds(base, K)])
                pltpu.sync_copy(bufc, gc_hbm.at[pl.ds(base, K)])
            return carry

        lax.fori_loop(0, ITERS, body, 0)

    return k(nodes_r, nodes_c, row, col)


def _sc_segsum(ef1d, row, col):
    E = row.shape[0]
    K = 128
    CH = E // K
    HALF = 5000
    ROWS = 5120
    ITERS = pl.cdiv(CH, 16)
    mesh = plsc.VectorSubcoreMesh(core_axis_name="c", subcore_axis_name="s")

    @functools.partial(
        pl.kernel, mesh=mesh,
        out_type=jax.ShapeDtypeStruct((2, 2, ROWS, 128), jnp.float32),
        scratch_types=[
            pltpu.VMEM((K,), jnp.int32), pltpu.VMEM((K,), jnp.int32),
            pltpu.VMEM((K * 16,), jnp.float32),
            pltpu.VMEM((K, 128), jnp.float32),
            pltpu.VMEM_SHARED((ROWS, 128), jnp.float32),
            pltpu.VMEM_SHARED((ROWS, 128), jnp.float32),
        ],
    )
    def k(ef_hbm, row_hbm, col_hbm, out_hbm, idxr, idxc, efbuf, pay,
          accs, acco):
        cid = lax.axis_index("c")
        sid = lax.axis_index("s")
        base = cid * HALF

        zero16 = jnp.zeros((16,), jnp.float32)
        ones_col = jnp.where(lax.iota(jnp.int32, 16) == 0, 1.0, 0.0)

        def pinit(r, carry):
            def cinit(j, carry2):
                pay[r, pl.ds(j * 16, 16)] = zero16
                return carry2
            lax.fori_loop(0, 8, cinit, 0)
            return carry

        lax.fori_loop(0, K, pinit, 0)

        def zbody(i, carry):
            c = sid + i * 16

            @pl.when(c < ROWS // K)
            def _():
                pltpu.sync_copy(pay, accs.at[pl.ds(c * K, K)])
                pltpu.sync_copy(pay, acco.at[pl.ds(c * K, K)])
            return carry

        lax.fori_loop(0, pl.cdiv(ROWS // K, 16), zbody, 0)

        def pones(r, carry):
            pay[r, pl.ds(16, 16)] = ones_col
            return carry

        lax.fori_loop(0, K, pones, 0)
        plsc.subcore_barrier()

        def body(i, carry):
            c = sid + i * 16

            @pl.when(c < CH)
            def _():
                off = c * K
                pltpu.sync_copy(row_hbm.at[pl.ds(off, K)], idxr)
                pltpu.sync_copy(col_hbm.at[pl.ds(off, K)], idxc)
                pltpu.sync_copy(ef_hbm.at[pl.ds(off * 16, K * 16)], efbuf)

                def fill(r, carry2):
                    pay[r, pl.ds(0, 16)] = efbuf[pl.ds(r * 16, 16)]
                    return carry2

                lax.fori_loop(0, K, fill, 0)

                def fix(j, carry2):
                    r = idxr[pl.ds(j * 16, 16)] - base
                    okr = (r >= 0) & (r < HALF)
                    idxr[pl.ds(j * 16, 16)] = jnp.where(okr, r, HALF)
                    cc = idxc[pl.ds(j * 16, 16)] - base
                    okc = (cc >= 0) & (cc < HALF)
                    idxc[pl.ds(j * 16, 16)] = jnp.where(okc, cc, HALF)
                    return carry2

                lax.fori_loop(0, K // 16, fix, 0)
                pltpu.sync_copy(pay, accs.at[idxr], add=True)
                pltpu.sync_copy(pay, acco.at[idxc], add=True)
            return carry

        lax.fori_loop(0, ITERS, body, 0)
        plsc.subcore_barrier()

        def wbody(i, carry):
            c = sid + i * 16

            @pl.when(c < ROWS // K)
            def _():
                pltpu.sync_copy(accs.at[pl.ds(c * K, K)],
                                out_hbm.at[0, cid, pl.ds(c * K, K)])
                pltpu.sync_copy(acco.at[pl.ds(c * K, K)],
                                out_hbm.at[1, cid, pl.ds(c * K, K)])
            return carry

        lax.fori_loop(0, pl.cdiv(ROWS // K, 16), wbody, 0)

    return k(ef1d, row, col)


def _edge_body(gr_ref, gc_ref, ef_ref, rev_ref,
               w1bt_ref, w1ct_ref, bne1_ref, w2t_ref, bne2_ref,
               geln_ref, beln_ref,
               wg1t_ref, bg1_ref, bng_ref, bnb_ref, wg2t_ref, bg2_ref,
               wet_ref, be_ref, m1t_ref, ba1_ref, m2t_ref, ba2_ref, s_ref,
               efo_ref, xxe_ref):
    ef = ef_ref[...]
    h = jax.nn.relu(jnp.dot(ef, wg1t_ref[...], preferred_element_type=jnp.float32)
                    + bg1_ref[...])
    h = h * (bng_ref[...] / jnp.sqrt(1.0 + 1e-5)) + bnb_ref[...]
    gates = jax.nn.sigmoid(jnp.dot(h, wg2t_ref[...], preferred_element_type=jnp.float32)
                           + bg2_ref[...])
    rev = gates * rev_ref[...]
    g1 = gr_ref[:, :160]
    gq = gr_ref[:, 160:288]
    g2 = gc_ref[:, :160]
    gv = gc_ref[:, 160:288]
    mid = g1 + g2 + bne1_ref[...]
    mid = mid + jnp.dot(ef, w1bt_ref[...], preferred_element_type=jnp.float32)
    mid = mid + jnp.dot(rev, w1ct_ref[...], preferred_element_type=jnp.float32)
    mid = jax.nn.relu(mid)
    efo = jnp.dot(mid, w2t_ref[...], preferred_element_type=jnp.float32) + bne2_ref[...]
    efo_ref[...] = _ln(efo, geln_ref[...], beln_ref[...])
    eh = jnp.dot(ef, wet_ref[...], preferred_element_type=jnp.float32) + be_ref[...]
    qe = jnp.concatenate([gq, eh], axis=1)
    a = jax.nn.relu(jnp.dot(qe, m1t_ref[...], preferred_element_type=jnp.float32)
                    + ba1_ref[...])
    p = jnp.dot(a, m2t_ref[...], preferred_element_type=jnp.float32) + ba2_ref[...]
    ex = jnp.exp(p)
    den = jnp.dot(ex, s_ref[...], preferred_element_type=jnp.float32)
    inv = jnp.dot(1.0 / den, s_ref[...].T, preferred_element_type=jnp.float32)
    xxe_ref[...] = ex * inv * gv


def _edge_compute(gr, gc, ef, rev, wdict):
    e = ef.shape[0]
    BLK = 512
    grid = (pl.cdiv(e, BLK),)
    full = lambda shp: pl.BlockSpec(shp, lambda i: tuple(0 for _ in shp))
    in_specs = [
        pl.BlockSpec((BLK, 384), lambda i: (i, 0)),
        pl.BlockSpec((BLK, 384), lambda i: (i, 0)),
        pl.BlockSpec((BLK, DE), lambda i: (i, 0)),
        pl.BlockSpec((BLK, DE), lambda i: (i, 0)),
        full((DE, 160)), full((DE, 160)), full((1, 160)),
        full((160, DE)), full((1, DE)),
        full((1, DE)), full((1, DE)),
        full((DE, 8)), full((1, 8)), full((1, 8)), full((1, 8)),
        full((8, 1)), full((1, 1)),
        full((DE, DE)), full((1, DE)),
        full((144, 144)), full((1, 144)),
        full((144, 128)), full((1, 128)),
        full((128, 4)),
    ]
    return pl.pallas_call(
        _edge_body,
        grid=grid,
        in_specs=in_specs,
        out_specs=[
            pl.BlockSpec((BLK, DE), lambda i: (i, 0)),
            pl.BlockSpec((BLK, DA), lambda i: (i, 0)),
        ],
        out_shape=[
            jax.ShapeDtypeStruct((e, DE), jnp.float32),
            jax.ShapeDtypeStruct((e, DA), jnp.float32),
        ],
        interpret=_INTERPRET,
    )(gr, gc, ef, rev, *wdict)


def _node_out_body(x_ref, agg_ref, ss_ref, os_ref,
                   wtt_ref, bt_ref, wp1t_ref, bp1_ref, wp2t_ref, bp2_ref,
                   gln_ref, bln_ref, o_ref):
    ds = ss_ref[:, 16:17]
    do = os_ref[:, 16:17]
    subj = jnp.where(ds > 0, ss_ref[:, :16] / jnp.maximum(ds, 1.0), 0.0)
    obj = jnp.where(do > 0, os_ref[:, :16] / jnp.maximum(do, 1.0), 0.0)
    twin = (jnp.dot(jnp.concatenate([subj, obj], axis=1), wtt_ref[...],
                    preferred_element_type=jnp.float32) + bt_ref[...])
    agg = jnp.where(ds > 0, agg_ref[...], 0.0)
    xx = jax.nn.relu(agg) * jax.nn.sigmoid(twin)
    cat = jnp.concatenate([x_ref[...], xx], axis=1)
    hh = jax.nn.relu(jnp.dot(cat, wp1t_ref[...], preferred_element_type=jnp.float32)
                     + bp1_ref[...])
    out = jnp.dot(hh, wp2t_ref[...], preferred_element_type=jnp.float32) + bp2_ref[...]
    o_ref[...] = _ln(out, gln_ref[...], bln_ref[...])


def _node_out(x, agg, ss, os_, wlist):
    n = x.shape[0]
    BLK = 512
    grid = (pl.cdiv(n, BLK),)
    full = lambda shp: pl.BlockSpec(shp, lambda i: tuple(0 for _ in shp))
    return pl.pallas_call(
        _node_out_body,
        grid=grid,
        in_specs=[
            pl.BlockSpec((BLK, DN), lambda i: (i, 0)),
            pl.BlockSpec((BLK, DA), lambda i: (i, 0)),
            pl.BlockSpec((BLK, 128), lambda i: (i, 0)),
            pl.BlockSpec((BLK, 128), lambda i: (i, 0)),
            full((32, 128)), full((1, 128)),
            full((256, 256)), full((1, 256)),
            full((256, 128)), full((1, 128)),
            full((1, 128)), full((1, 128)),
        ],
        out_specs=pl.BlockSpec((BLK, DN), lambda i: (i, 0)),
        out_shape=jax.ShapeDtypeStruct((n, DN), jnp.float32),
        interpret=_INTERPRET,
    )(x, agg, ss, os_, *wlist)


def kernel(x, edge_feature, edge_index, W_ne1, b_ne1, W_ne2, b_ne2, g_eln,
           b_eln, Wg1, bg1, bn_g, bn_b, Wg2, bg2, Wv, bv, Wq, bq, We, be,
           Wa1, ba1, Wa2, ba2, Wt, bt, Wp1, bp1, Wp2, bp2, g_ln, b_ln):
    N = x.shape[0]
    E = edge_index.shape[1]
    row = edge_index[0].astype(jnp.int32)
    col = edge_index[1].astype(jnp.int32)

    keys = row.astype(jnp.int64) * N + col.astype(jnp.int64)
    rkeys = col.astype(jnp.int64) * N + row.astype(jnp.int64)
    order = jnp.argsort(keys)
    sk = keys[order]
    pos = jnp.clip(jnp.searchsorted(sk, rkeys), 0, E - 1)
    found = sk[pos] == rkeys
    rev_idx = order[pos]
    rev_raw = jnp.where(found[:, None], edge_feature[rev_idx], 0.0)

    z96 = jnp.zeros((DN, 96), jnp.float32)
    M = jnp.concatenate([
        W_ne1[:, :DN].T, Wq.T, z96, W_ne1[:, 160:288].T, Wv.T, z96], axis=1)
    bnode = jnp.concatenate([
        jnp.zeros((160,), jnp.float32), bq, jnp.zeros((96,), jnp.float32),
        jnp.zeros((160,), jnp.float32), bv,
        jnp.zeros((96,), jnp.float32)])[None, :]
    nodes_r, nodes_c = _node_precompute(x, M, bnode)
    gr, gc = _sc_gather2(nodes_r, nodes_c, row, col)

    eye4 = jnp.eye(4, dtype=jnp.float32)
    wlist = (
        W_ne1[:, 128:144].T, W_ne1[:, 144:160].T, b_ne1[None, :],
        W_ne2.T, b_ne2[None, :], g_eln[None, :], b_eln[None, :],
        Wg1.T, bg1[None, :], bn_g[None, :], bn_b[None, :],
        Wg2.T, bg2[None, :],
        We.T, be[None, :],
        jnp.kron(Wa1, eye4).T, jnp.kron(ba1, jnp.ones(4, jnp.float32))[None, :],
        jnp.kron(Wa2, eye4).T, jnp.kron(ba2, jnp.ones(4, jnp.float32))[None, :],
        jnp.tile(eye4, (32, 1)),
    )
    ef_out, xx_e = _edge_compute(gr, gc, edge_feature, rev_raw, wlist)

    agg = jax.ops.segment_max(xx_e, row, num_segments=N)
    seg = _sc_segsum(ef_out.reshape(E * DE), row, col)
    subj32 = jnp.concatenate([seg[0, 0, :5000], seg[0, 1, :5000]], axis=0)
    obj32 = jnp.concatenate([seg[1, 0, :5000], seg[1, 1, :5000]], axis=0)

    wlist3 = (
        Wt.T, bt[None, :], Wp1.T, bp1[None, :], Wp2.T, bp2[None, :],
        g_ln[None, :], b_ln[None, :],
    )
    out = _node_out(x, agg, subj32, obj32, wlist3)
    return (out, ef_out)
